# Initial kernel scaffold; baseline (speedup 1.0000x reference)
import jax
import jax.numpy as jnp
from jax.experimental import pallas as pl


def kernel(x, edge_index, params):
    raise NotImplementedError("write your pallas kernel here")


# speedup vs baseline: 1.5778x; 1.5778x over previous
import functools

import jax
import jax.numpy as jnp
from jax.experimental import pallas as pl


def _proj_kernel(h_ref, w_ref, b_ref, o_ref):
    o_ref[...] = (
        jnp.dot(h_ref[...], w_ref[...], preferred_element_type=jnp.float32)
        + b_ref[...]
    )


def _proj(h, W4, b4):
    n, d = h.shape
    do = W4.shape[1]
    return pl.pallas_call(
        _proj_kernel,
        out_shape=jax.ShapeDtypeStruct((n, do), jnp.float32),
    )(h, W4, b4)


def kernel(x, edge_index, params):
    src = edge_index[0]
    dst = edge_index[1]
    n = x.shape[0]
    h = x
    for i, p in enumerate(params):
        W4 = jnp.concatenate([p["Wq"], p["Wk"], p["Wv"], p["Ws"]], axis=1)
        b4 = jnp.concatenate([p["bq"], p["bk"], p["bv"], p["bs"]])[None, :]
        qkvs = _proj(h, W4, b4)
        d = h.shape[1]
        q, k, v, s = (
            qkvs[:, :d],
            qkvs[:, d : 2 * d],
            qkvs[:, 2 * d : 3 * d],
            qkvs[:, 3 * d :],
        )
        alpha = jnp.sum(q[dst] * k[src], axis=-1) / jnp.sqrt(float(d))
        ex = jnp.exp(alpha)
        denom = jax.ops.segment_sum(ex, dst, num_segments=n)
        w = ex / (denom[dst] + 1e-16)
        out = jax.ops.segment_sum(v[src] * w[:, None], dst, num_segments=n)
        h = out + s
        if i < len(params) - 1:
            h = jax.nn.relu(h)
    return h


# speedup vs baseline: 7.5161x; 4.7638x over previous
import dataclasses
import functools

import jax
import jax.numpy as jnp
from jax import lax
from jax.experimental import pallas as pl
from jax.experimental.pallas import tpu as pltpu
from jax.experimental.pallas import tpu_sc as plsc

NC = 2
NS = 16
L = 16
NW = NC * NS


def _proj_body(h_ref, w_ref, b_ref, q_ref, k_ref, v_ref, s_ref):
    res = (
        jnp.dot(h_ref[...], w_ref[...], preferred_element_type=jnp.float32)
        + b_ref[...]
    )
    d = q_ref.shape[-1]
    q_ref[...] = res[:, :d]
    k_ref[...] = res[:, d : 2 * d]
    v_ref[...] = res[:, 2 * d : 3 * d]
    s_ref[...] = res[:, 3 * d :]


def _proj(h, W4, b4):
    n, d = h.shape
    blk = 2000
    out = jax.ShapeDtypeStruct((n, d), jnp.float32)
    return pl.pallas_call(
        _proj_body,
        grid=(n // blk,),
        in_specs=[
            pl.BlockSpec((blk, d), lambda i: (i, 0)),
            pl.BlockSpec((d, 4 * d), lambda i: (0, 0)),
            pl.BlockSpec((1, 4 * d), lambda i: (0, 0)),
        ],
        out_specs=[pl.BlockSpec((blk, d), lambda i: (i, 0))] * 4,
        out_shape=[out, out, out, out],
    )(h, W4, b4)


def _alpha_body(q_ref, k_ref, o_ref, *, scale):
    o_ref[...] = (
        lax.dot_general(
            q_ref[...],
            k_ref[...],
            (((1,), (1,)), ((), ())),
            preferred_element_type=jnp.float32,
        )
        * scale
    )


def _alpha(q, k):
    n, d = q.shape
    bi = 400
    return pl.pallas_call(
        functools.partial(_alpha_body, scale=1.0 / (float(d) ** 0.5)),
        grid=(n // bi,),
        in_specs=[
            pl.BlockSpec((bi, d), lambda i: (i, 0)),
            pl.BlockSpec((n, d), lambda i: (0, 0)),
        ],
        out_specs=pl.BlockSpec((bi, n), lambda i: (i, 0)),
        out_shape=jax.ShapeDtypeStruct((n, n), jnp.float32),
    )(q, k)


def _edge_call(atbl, vtbl, src_e, dst_e, zz, n, d, e):
    epw = e // NW
    C = 80
    nchunk = epw // C
    npad = zz.shape[0]
    rpt = npad // NS

    mesh = plsc.VectorSubcoreMesh(
        core_axis_name="c", subcore_axis_name="s", num_cores=NC,
        num_subcores=NS,
    )

    cp = pltpu.CompilerParams()
    if "needs_layout_passes" in pltpu.CompilerParams.__dataclass_fields__:
        cp = dataclasses.replace(cp, needs_layout_passes=False)

    @functools.partial(
        pl.kernel,
        compiler_params=cp,
        out_type=[
            jax.ShapeDtypeStruct((NC, npad, d), jnp.float32),
            jax.ShapeDtypeStruct((NW, 1, n), jnp.float32),
        ],
        mesh=mesh,
        scratch_types=[
            pltpu.VMEM((C,), jnp.int32),
            pltpu.VMEM((C,), jnp.int32),
            pltpu.VMEM((C,), jnp.int32),
            pltpu.VMEM((C,), jnp.int32),
            pltpu.VMEM((C,), jnp.float32),
            pltpu.VMEM((C, 128), jnp.float32),
            pltpu.VMEM((C, d), jnp.float32),
            pltpu.VMEM((1, n), jnp.float32),
            pltpu.VMEM_SHARED((npad, d), jnp.float32),
            pltpu.SemaphoreType.DMA,
            pltpu.SemaphoreType.DMA,
        ],
    )
    def edge_kernel(atbl_hbm, vtbl_hbm, src_hbm, dst_hbm, zz_hbm,
                    acc_hbm, den_hbm,
                    src_b, dst_b, rid_b, cid_b, ex_b, arows_b, vrows_b,
                    den_t, acc_s, sem_v, sem_a):
        cid = lax.axis_index("c")
        sid = lax.axis_index("s")
        wid = cid * NS + sid

        @pl.loop(0, n, step=L)
        def _zden(i):
            den_t[0, pl.ds(i, L)] = jnp.zeros((L,), jnp.float32)

        row0 = pl.multiple_of(sid * rpt, rpt)
        pltpu.sync_copy(zz_hbm.at[pl.ds(row0, rpt)], acc_s.at[pl.ds(row0, rpt)])
        plsc.subcore_barrier()

        base = pl.multiple_of(wid * epw, epw)

        @pl.loop(0, nchunk)
        def _chunk(c):
            off = base + c * C
            pltpu.sync_copy(src_hbm.at[pl.ds(off, C)], src_b)
            pltpu.sync_copy(dst_hbm.at[pl.ds(off, C)], dst_b)
            vcp = pltpu.async_copy(vtbl_hbm.at[src_b], vrows_b, sem_v)

            @pl.loop(0, C, step=L)
            def _ids(i):
                s = src_b[pl.ds(i, L)]
                t = dst_b[pl.ds(i, L)]
                flat = t * n + s
                rid_b[pl.ds(i, L)] = lax.shift_right_logical(flat, 7)
                cid_b[pl.ds(i, L)] = lax.bitwise_and(flat, 127)

            pltpu.async_copy(atbl_hbm.at[rid_b], arows_b, sem_a).wait()

            @pl.loop(0, C // L)
            def _soft(g):
                rows = lax.iota(jnp.int32, L) + g * L
                cols = cid_b[pl.ds(g * L, L)]
                a = plsc.load_gather(arows_b, [rows, cols])
                ex = jnp.exp(a)
                plsc.addupdate_scatter(
                    den_t,
                    [jnp.zeros((L,), jnp.int32), dst_b[pl.ds(g * L, L)]],
                    ex,
                )
                ex_b[pl.ds(g * L, L)] = ex

            vcp.wait()

            @pl.loop(0, C // L)
            def _scale(g):
                for eloc in range(L):
                    ei_ = g * L + eloc
                    w = plsc.load_gather(
                        ex_b, [jnp.zeros((L,), jnp.int32) + ei_]
                    )
                    for j in range(d // L):
                        sl = pl.ds(j * L, L)
                        vrows_b[ei_, sl] = vrows_b[ei_, sl] * w

            pltpu.sync_copy(vrows_b, acc_s.at[dst_b], add=True)

        pltpu.sync_copy(den_t, den_hbm.at[wid])
        plsc.subcore_barrier()
        pltpu.sync_copy(
            acc_s.at[pl.ds(row0, rpt)],
            acc_hbm.at[cid, pl.ds(row0, rpt)],
        )

    return edge_kernel(atbl, vtbl, src_e, dst_e, zz)


def _combine_body(acc_ref, den_ref, s_ref, o_ref, *, relu):
    den = jnp.sum(den_ref[...], axis=1)
    h = (acc_ref[0] + acc_ref[1]) / (den[:, None] + 1e-16) + s_ref[...]
    if relu:
        h = jnp.maximum(h, 0.0)
    o_ref[...] = h


def _combine(acc, den, s, relu):
    n, d = s.shape
    blk = 2000
    return pl.pallas_call(
        functools.partial(_combine_body, relu=relu),
        grid=(n // blk,),
        in_specs=[
            pl.BlockSpec((NC, blk, d), lambda i: (0, i, 0)),
            pl.BlockSpec((blk, NW), lambda i: (i, 0)),
            pl.BlockSpec((blk, d), lambda i: (i, 0)),
        ],
        out_specs=pl.BlockSpec((blk, d), lambda i: (i, 0)),
        out_shape=jax.ShapeDtypeStruct((n, d), jnp.float32),
    )(acc, den, s)


def kernel(x, edge_index, params):
    n, d = x.shape
    e = edge_index.shape[1]
    npad = ((n + NS * 8 - 1) // (NS * 8)) * (NS * 8)
    zz = jnp.zeros((npad, d), jnp.float32)
    src_e = edge_index[0]
    dst_e = edge_index[1]
    h = x
    for i, p in enumerate(params):
        W4 = jnp.concatenate([p["Wq"], p["Wk"], p["Wv"], p["Ws"]], axis=1)
        b4 = jnp.concatenate([p["bq"], p["bk"], p["bv"], p["bs"]])[None, :]
        q, k, v, s = _proj(h, W4, b4)
        atbl = _alpha(q, k).reshape(n * n // 128, 128)
        acc, den = _edge_call(atbl, v, src_e, dst_e, zz, n, d, e)
        h = _combine(acc, den.reshape(NW, n).T, s, relu=i < len(params) - 1)
    return h


# speedup vs baseline: 12.9664x; 1.7251x over previous
import dataclasses
import functools

import jax
import jax.numpy as jnp
from jax import lax
from jax.experimental import pallas as pl
from jax.experimental.pallas import tpu as pltpu
from jax.experimental.pallas import tpu_sc as plsc

NC = 2
NS = 16
L = 16
NW = NC * NS
SG = 8


def _proj_body(h_ref, w_ref, b_ref, q_ref, k_ref, v_ref, s_ref):
    res = (
        jnp.dot(h_ref[...], w_ref[...], preferred_element_type=jnp.float32)
        + b_ref[...]
    )
    d = q_ref.shape[-1]
    q_ref[...] = res[:, :d]
    k_ref[...] = res[:, d : 2 * d]
    v_ref[...] = res[:, 2 * d : 3 * d]
    s_ref[...] = res[:, 3 * d :]


def _proj(h, W4, b4):
    n, d = h.shape
    blk = 2000
    out = jax.ShapeDtypeStruct((n, d), jnp.float32)
    return pl.pallas_call(
        _proj_body,
        grid=(n // blk,),
        in_specs=[
            pl.BlockSpec((blk, d), lambda i: (i, 0)),
            pl.BlockSpec((d, 4 * d), lambda i: (0, 0)),
            pl.BlockSpec((1, 4 * d), lambda i: (0, 0)),
        ],
        out_specs=[pl.BlockSpec((blk, d), lambda i: (i, 0))] * 4,
        out_shape=[out, out, out, out],
    )(h, W4, b4)


def _edge_call(qtbl, ktbl, vtbl, src2, dst2, n, d, e):
    epw = e // NW
    C = 16
    upt = epw // C
    npad = ((n + NS * 8 - 1) // (NS * 8)) * (NS * 8)
    rpt = npad // NS
    dn = n // 128 if n % 128 == 0 else n // 128 + 1
    scale = 1.0 / (float(d) ** 0.5)
    nloop = (upt - 5) // 3

    mesh = plsc.VectorSubcoreMesh(
        core_axis_name="c", subcore_axis_name="s", num_cores=NC,
        num_subcores=NS,
    )

    cp = pltpu.CompilerParams()
    if "needs_layout_passes" in pltpu.CompilerParams.__dataclass_fields__:
        cp = dataclasses.replace(cp, needs_layout_passes=False)

    @functools.partial(
        pl.kernel,
        compiler_params=cp,
        out_type=[
            jax.ShapeDtypeStruct((NC, npad, d), jnp.float32),
            jax.ShapeDtypeStruct((NW, dn, 128), jnp.float32),
        ],
        mesh=mesh,
        scratch_types=[
            pltpu.VMEM((1, epw), jnp.int32),
            pltpu.VMEM((1, epw), jnp.int32),
            [pltpu.VMEM((C, d), jnp.float32) for _ in range(3)],
            [pltpu.VMEM((C, d), jnp.float32) for _ in range(3)],
            [pltpu.VMEM((C, d), jnp.float32) for _ in range(3)],
            [pltpu.VMEM((C,), jnp.int32) for _ in range(3)],
            pltpu.VMEM((L,), jnp.float32),
            pltpu.VMEM((L,), jnp.float32),
            pltpu.VMEM((dn, 128), jnp.float32),
            pltpu.VMEM_SHARED((npad, d), jnp.float32),
            [pltpu.SemaphoreType.DMA for _ in range(3)],
            [pltpu.SemaphoreType.DMA for _ in range(3)],
            pltpu.SemaphoreType.DMA,
        ],
    )
    def edge_kernel(qt, kt, vt, src_hbm, dst_hbm, acc_hbm, den_hbm,
                    src_a, dst_a, qb, kb, vb, db, ab, exb, den_t, acc_s,
                    semg, sems, semi):
        cid = lax.axis_index("c")
        sid = lax.axis_index("s")
        wid = cid * NS + sid

        pltpu.async_copy(src_hbm.at[wid], src_a, semi).wait()
        pltpu.async_copy(dst_hbm.at[wid], dst_a, semi).wait()

        @pl.loop(0, dn)
        def _zden(i):
            for j in range(128 // L):
                den_t[i, pl.ds(j * L, L)] = jnp.zeros((L,), jnp.float32)

        @pl.loop(0, C)
        def _zvb(i):
            for j in range(d // L):
                vb[2][i, pl.ds(j * L, L)] = jnp.zeros((L,), jnp.float32)

        row0 = pl.multiple_of(sid * rpt, 8)

        @pl.loop(0, rpt, step=8)
        def _zacc(i):
            pltpu.sync_copy(
                vb[2].at[pl.ds(0, 8)],
                acc_s.at[pl.ds(pl.multiple_of(row0 + i, 8), 8)],
            )
        plsc.subcore_barrier()

        def issue(u, b):
            sidx = src_a.at[0, pl.ds(u * C, C)]
            didx = dst_a.at[0, pl.ds(u * C, C)]
            pltpu.async_copy(qt.at[didx], qb[b], semg[b])
            pltpu.async_copy(kt.at[sidx], kb[b], semg[b])
            pltpu.async_copy(vt.at[sidx], vb[b], semg[b])

        def wait_gathers(u, b):
            sidx = src_a.at[0, pl.ds(u * C, C)]
            didx = dst_a.at[0, pl.ds(u * C, C)]
            pltpu.make_async_copy(qt.at[didx], qb[b], semg[b]).wait()
            pltpu.make_async_copy(kt.at[sidx], kb[b], semg[b]).wait()
            pltpu.make_async_copy(vt.at[sidx], vb[b], semg[b]).wait()

        def scatter(b):
            pltpu.async_copy(vb[b], acc_s.at[db[b]], sems[b], add=True)

        def wait_scatter(b):
            pltpu.make_async_copy(vb[b], acc_s.at[db[b]], sems[b]).wait()

        lane0 = lax.iota(jnp.int32, L) == 0

        def compute(u, b):
            @pl.loop(0, C)
            def _dot(eloc):
                part = qb[b][eloc, pl.ds(0, L)] * kb[b][eloc, pl.ds(0, L)]
                for j in range(1, d // L):
                    sl = pl.ds(j * L, L)
                    part = part + qb[b][eloc, sl] * kb[b][eloc, sl]
                a = jnp.sum(part) * scale
                plsc.store_scatter(
                    ab,
                    [jnp.zeros((L,), jnp.int32) + eloc],
                    jnp.zeros((L,), jnp.float32) + a,
                    mask=lane0,
                )

            dstv = dst_a[0, pl.ds(u * C, C)]
            ex = jnp.exp(ab[...])
            plsc.addupdate_scatter(
                den_t,
                [lax.shift_right_logical(dstv, 7),
                 lax.bitwise_and(dstv, 127)],
                ex,
            )
            db[b][...] = dstv
            exb[...] = ex

            @pl.loop(0, C)
            def _scl(eloc):
                w = plsc.load_gather(exb, [jnp.zeros((L,), jnp.int32) + eloc])
                for j in range(d // L):
                    sl = pl.ds(j * L, L)
                    vb[b][eloc, sl] = vb[b][eloc, sl] * w

        def slot(u, b, wait_prev, issue_next):
            wait_gathers(u, b)
            compute(u, b)
            scatter(b)
            if issue_next:
                b2 = (b + 2) % 3
                if wait_prev:
                    wait_scatter(b2)
                issue(u + 2, b2)

        issue(0, 0)
        issue(1, 1)
        slot(0, 0, False, True)
        slot(1, 1, True, True)
        slot(2, 2, True, True)

        @pl.loop(1, nloop + 1)
        def _main(i):
            u = 3 * i
            slot(u, 0, True, True)
            slot(u + 1, 1, True, True)
            slot(u + 2, 2, True, True)

        tails = [(u, u % 3) for u in range(3 * (nloop + 1), upt)]
        for idx, (u, b) in enumerate(tails):
            slot(u, b, True, idx < len(tails) - 2)
        for u in range(upt - 3, upt):
            wait_scatter(u % 3)

        pltpu.sync_copy(den_t, den_hbm.at[wid])
        plsc.subcore_barrier()
        pltpu.sync_copy(
            acc_s.at[pl.ds(row0, rpt)],
            acc_hbm.at[cid, pl.ds(row0, rpt)],
        )

    return edge_kernel(qtbl, ktbl, vtbl, src2, dst2)


def _combine_body(acc_ref, den_ref, s_ref, o_ref, *, relu):
    den = jnp.sum(den_ref[...], axis=1)
    h = (acc_ref[0] + acc_ref[1]) / (den[:, None] + 1e-16) + s_ref[...]
    if relu:
        h = jnp.maximum(h, 0.0)
    o_ref[...] = h


def _combine(acc, den, s, relu):
    n, d = s.shape
    blk = 2000
    return pl.pallas_call(
        functools.partial(_combine_body, relu=relu),
        grid=(n // blk,),
        in_specs=[
            pl.BlockSpec((NC, blk, d), lambda i: (0, i, 0)),
            pl.BlockSpec((blk, NW), lambda i: (i, 0)),
            pl.BlockSpec((blk, d), lambda i: (i, 0)),
        ],
        out_specs=pl.BlockSpec((blk, d), lambda i: (i, 0)),
        out_shape=jax.ShapeDtypeStruct((n, d), jnp.float32),
    )(acc, den, s)


def kernel(x, edge_index, params):
    n, d = x.shape
    e = edge_index.shape[1]
    src2 = edge_index[0].reshape(NW, 1, e // NW)
    dst2 = edge_index[1].reshape(NW, 1, e // NW)
    h = x
    for i, p in enumerate(params):
        W4 = jnp.concatenate([p["Wq"], p["Wk"], p["Wv"], p["Ws"]], axis=1)
        b4 = jnp.concatenate([p["bq"], p["bk"], p["bv"], p["bs"]])[None, :]
        q, k, v, s = _proj(h, W4, b4)
        acc, den = _edge_call(q, k, v, src2, dst2, n, d, e)
        h = _combine(acc, den.reshape(NW, -1).T, s, relu=i < len(params) - 1)
    return h
